# baseline (speedup 1.0000x reference)
import jax
import jax.numpy as jnp
from jax import lax
from jax.experimental import pallas as pl
from jax.experimental.pallas import tpu as pltpu
from jax.experimental.pallas import tpu_sc as plsc

NC = 2
NS = 16
L = 16
NW = NC * NS

B = 16384
D = 64
BPW = B // NW
GCHUNK = 128
NCHUNK = BPW // GCHUNK

_SC_PARAMS = pltpu.CompilerParams(use_tc_tiling_on_sc=False)


def _wid():
    return lax.axis_index("s") * NC + lax.axis_index("c")


def _gather_body(idx_hbm, tab_hbm, out_hbm, idx_v, rows_v, sem):
    base = _wid() * BPW
    pltpu.sync_copy(idx_hbm.at[pl.ds(base, BPW)], idx_v)
    handles = []
    for j in range(NCHUNK):
        sl = pl.ds(j * GCHUNK, GCHUNK)
        handles.append(pltpu.async_copy(tab_hbm.at[idx_v.at[sl]], rows_v.at[sl], sem))
    for h in handles:
        h.wait()
    pltpu.sync_copy(rows_v, out_hbm.at[pl.ds(base, BPW)])


def _dot_body(u_hbm, i_hbm, out_hbm, u_rows, i_rows, out_v, semu, semi):
    base = _wid() * BPW
    hu = pltpu.async_copy(u_hbm.at[pl.ds(base, BPW)], u_rows, semu)
    hi = pltpu.async_copy(i_hbm.at[pl.ds(base, BPW)], i_rows, semi)
    hu.wait()
    hi.wait()

    lanes = lax.iota(jnp.int32, L)
    perms = {h: lanes ^ h for h in (8, 4, 2, 1)}
    masks = {h: (lanes & h) != 0 for h in (8, 4, 2, 1)}

    def lperm(v, h):
        return v.at[perms[h]].get(mode="promise_in_bounds", unique_indices=True)

    def group(g, _):
        vs = []
        for rl in range(L):
            r = g * L + rl
            acc = u_rows[r, pl.ds(0, L)] * i_rows[r, pl.ds(0, L)]
            for k in range(1, D // L):
                acc += u_rows[r, pl.ds(k * L, L)] * i_rows[r, pl.ds(k * L, L)]
            vs.append(acc)
        for h in (8, 4, 2, 1):
            half = len(vs) // 2
            vs = [jnp.where(masks[h],
                            vs[q + half] + lperm(vs[q + half], h),
                            vs[q] + lperm(vs[q], h))
                  for q in range(half)]
        out_v[pl.ds(g * L, L)] = vs[0]
        return 0

    lax.fori_loop(0, BPW // L, group, 0)
    pltpu.sync_copy(out_v, out_hbm.at[pl.ds(base, BPW)])


def _mesh():
    return plsc.VectorSubcoreMesh(core_axis_name="c", subcore_axis_name="s")


def _gather(idx, table):
    return pl.kernel(
        _gather_body,
        out_type=jax.ShapeDtypeStruct((B, D), jnp.float32),
        mesh=_mesh(),
        compiler_params=_SC_PARAMS,
        scratch_types=[
            pltpu.VMEM((BPW,), jnp.int32),
            pltpu.VMEM((BPW, D), jnp.float32),
            pltpu.SemaphoreType.DMA,
        ],
    )(idx, table)


@jax.jit
def _mf_dot(user_id, item_id, user_table, item_table):
    u_rows = _gather(user_id, user_table)
    i_rows = _gather(item_id, item_table)
    return pl.kernel(
        _dot_body,
        out_type=jax.ShapeDtypeStruct((B,), jnp.float32),
        mesh=_mesh(),
        compiler_params=_SC_PARAMS,
        scratch_types=[
            pltpu.VMEM((BPW, D), jnp.float32),
            pltpu.VMEM((BPW, D), jnp.float32),
            pltpu.VMEM((BPW,), jnp.float32),
            pltpu.SemaphoreType.DMA,
            pltpu.SemaphoreType.DMA,
        ],
    )(u_rows, i_rows)


def kernel(user_id, item_id, user_table, item_table):
    return _mf_dot(user_id, item_id, user_table, item_table)
